# baseline (speedup 1.0000x reference)
import functools

import jax
import jax.numpy as jnp
from jax import lax
from jax.experimental import pallas as pl
from jax.experimental.pallas import tpu as pltpu
from jax.experimental.pallas import tpu_sc as plsc

_LANES = 16


def _build_gather(P, C, K, R):
    info = plsc.get_sparse_core_info()
    nc, ns = info.num_cores, info.num_subcores
    nw = nc * ns
    rows_w = P // nw
    n_chunks = rows_w // R
    nj = K // _LANES
    assert P % nw == 0 and rows_w % R == 0 and K % _LANES == 0

    mesh = plsc.VectorSubcoreMesh(core_axis_name="c", subcore_axis_name="s")

    @functools.partial(
        pl.kernel,
        mesh=mesh,
        out_type=jax.ShapeDtypeStruct((P * K,), jnp.float32),
        compiler_params=pltpu.CompilerParams(needs_layout_passes=False),
        scratch_types=[
            pltpu.VMEM((K,), jnp.int32),
            pltpu.VMEM((2 * R * C,), jnp.float32),
            pltpu.VMEM((2 * R * K,), jnp.float32),
            pltpu.SemaphoreType.DMA,
            pltpu.SemaphoreType.DMA,
        ],
    )
    def gather_rows(x_hbm, keep_hbm, out_hbm, keep_v, ibufs, obufs, isem, osem):
        wid = lax.axis_index("s") * nc + lax.axis_index("c")
        row0 = wid * rows_w
        pltpu.sync_copy(keep_hbm, keep_v)
        kvecs = [keep_v[pl.ds(j * _LANES, _LANES)] for j in range(nj)]

        def in_slice(k):
            return x_hbm.at[pl.ds((row0 + k * R) * C, R * C)]

        def out_slice(k):
            return out_hbm.at[pl.ds((row0 + k * R) * K, R * K)]

        pltpu.async_copy(in_slice(0), ibufs.at[pl.ds(0, R * C)], isem)

        def step(k):
            p = lax.rem(k, 2)
            ib = ibufs.at[pl.ds(p * (R * C), R * C)]
            ob = obufs.at[pl.ds(p * (R * K), R * K)]

            @pl.when(k + 1 < n_chunks)
            def _():
                pltpu.async_copy(
                    in_slice(k + 1),
                    ibufs.at[pl.ds((1 - p) * (R * C), R * C)],
                    isem,
                )

            pltpu.make_async_copy(in_slice(k), ib, isem).wait()

            @pl.when(k >= 2)
            def _():
                pltpu.make_async_copy(ob, out_slice(k - 2), osem).wait()

            ibase = p * (R * C)

            def row(r, idxs):
                obase = p * (R * K) + r * K
                for j in range(nj):
                    obufs[pl.ds(obase + j * _LANES, _LANES)] = (
                        plsc.load_gather(ibufs, [idxs[j]])
                    )
                return tuple(ix + C for ix in idxs)

            pl.loop(
                0,
                R,
                init_carry=tuple(kv + ibase for kv in kvecs),
                unroll=4,
            )(row)
            pltpu.async_copy(ob, out_slice(k), osem)

        pl.loop(0, n_chunks)(step)

        for k in (n_chunks - 2, n_chunks - 1):
            pltpu.make_async_copy(
                obufs.at[pl.ds((k % 2) * (R * K), R * K)], out_slice(k), osem
            ).wait()

    return gather_rows


def kernel(x, keep):
    B, C, H, W = x.shape
    K = keep.shape[0]
    P = B * H * W
    xt = x.transpose(0, 2, 3, 1).reshape(P * C)
    gather_rows = _build_gather(P, C, K, R=56)
    out_t = gather_rows(xt, keep)
    return out_t.reshape(B, H, W, K).transpose(0, 3, 1, 2)


# speedup vs baseline: 2.9411x; 2.9411x over previous
import functools

import jax
import jax.numpy as jnp
from jax import lax
from jax.experimental import pallas as pl
from jax.experimental.pallas import tpu as pltpu
from jax.experimental.pallas import tpu_sc as plsc

_LANES = 16


def _build_gather(P, C, K, rows_chunk):
    info = plsc.get_sparse_core_info()
    nc, ns = info.num_cores, info.num_subcores
    nw = nc * ns
    rows_w = P // nw
    n_chunks = rows_w // rows_chunk
    assert P % nw == 0 and rows_w % rows_chunk == 0 and K % _LANES == 0

    mesh = plsc.VectorSubcoreMesh(core_axis_name="c", subcore_axis_name="s")

    @functools.partial(
        pl.kernel,
        mesh=mesh,
        out_type=jax.ShapeDtypeStruct((P, K), jnp.float32),
        compiler_params=pltpu.CompilerParams(needs_layout_passes=False),
        scratch_types=[
            pltpu.VMEM((K,), jnp.int32),
            pltpu.VMEM((2, rows_chunk, C), jnp.float32),
            pltpu.VMEM((2, rows_chunk, K), jnp.float32),
            pltpu.SemaphoreType.DMA,
            pltpu.SemaphoreType.DMA,
        ],
    )
    def gather_rows(x_hbm, keep_hbm, out_hbm, keep_v, ibufs, obufs, isem, osem):
        wid = lax.axis_index("s") * nc + lax.axis_index("c")
        row0 = wid * rows_w
        pltpu.sync_copy(keep_hbm, keep_v)
        kvecs = [keep_v[pl.ds(j * _LANES, _LANES)] for j in range(K // _LANES)]

        def in_slice(k):
            return x_hbm.at[pl.ds(row0 + k * rows_chunk, rows_chunk)]

        def out_slice(k):
            return out_hbm.at[pl.ds(row0 + k * rows_chunk, rows_chunk)]

        pltpu.async_copy(in_slice(0), ibufs.at[0], isem)

        def step(k):
            p = lax.rem(k, 2)

            @pl.when(k + 1 < n_chunks)
            def _():
                pltpu.async_copy(in_slice(k + 1), ibufs.at[1 - p], isem)

            pltpu.make_async_copy(in_slice(k), ibufs.at[p], isem).wait()

            @pl.when(k >= 2)
            def _():
                pltpu.make_async_copy(
                    obufs.at[p], out_slice(k - 2), osem
                ).wait()

            pv = jnp.broadcast_to(p, (_LANES,)).astype(jnp.int32)

            def row(r):
                rv = jnp.broadcast_to(r, (_LANES,)).astype(jnp.int32)
                for j in range(K // _LANES):
                    obufs[p, r, pl.ds(j * _LANES, _LANES)] = plsc.load_gather(
                        ibufs, [pv, rv, kvecs[j]]
                    )

            pl.loop(0, rows_chunk, unroll=8)(row)
            pltpu.async_copy(obufs.at[p], out_slice(k), osem)

        pl.loop(0, n_chunks)(step)

        for k in (n_chunks - 2, n_chunks - 1):
            pltpu.make_async_copy(obufs.at[k % 2], out_slice(k), osem).wait()

    return gather_rows


def kernel(x, keep):
    B, C, H, W = x.shape
    K = keep.shape[0]
    P = B * H * W
    xt = x.transpose(0, 2, 3, 1).reshape(P, C)
    gather_rows = _build_gather(P, C, K, rows_chunk=56)
    out_t = gather_rows(xt, keep)
    return out_t.reshape(B, H, W, K).transpose(0, 3, 1, 2)


# speedup vs baseline: 3.0636x; 1.0417x over previous
import functools

import jax
import jax.numpy as jnp
from jax import lax
from jax.experimental import pallas as pl
from jax.experimental.pallas import tpu as pltpu
from jax.experimental.pallas import tpu_sc as plsc

_LANES = 16


def _build_gather(P, C, K, rows_chunk):
    info = plsc.get_sparse_core_info()
    nc, ns = info.num_cores, info.num_subcores
    nw = nc * ns
    rows_w = P // nw
    n_chunks = rows_w // rows_chunk
    assert P % nw == 0 and rows_w % rows_chunk == 0 and K % _LANES == 0

    mesh = plsc.VectorSubcoreMesh(core_axis_name="c", subcore_axis_name="s")

    @functools.partial(
        pl.kernel,
        mesh=mesh,
        out_type=jax.ShapeDtypeStruct((P, K), jnp.float32),
        compiler_params=pltpu.CompilerParams(needs_layout_passes=False),
        scratch_types=[
            pltpu.VMEM((K,), jnp.int32),
            pltpu.VMEM((2 * rows_chunk, C), jnp.float32),
            pltpu.VMEM((2 * rows_chunk, K), jnp.float32),
            pltpu.SemaphoreType.DMA,
            pltpu.SemaphoreType.DMA,
        ],
    )
    def gather_rows(x_hbm, keep_hbm, out_hbm, keep_v, ibufs, obufs, isem, osem):
        wid = lax.axis_index("s") * nc + lax.axis_index("c")
        row0 = wid * rows_w
        pltpu.sync_copy(keep_hbm, keep_v)
        kvecs = [keep_v[pl.ds(j * _LANES, _LANES)] for j in range(K // _LANES)]

        def in_slice(k):
            return x_hbm.at[pl.ds(row0 + k * rows_chunk, rows_chunk)]

        def out_slice(k):
            return out_hbm.at[pl.ds(row0 + k * rows_chunk, rows_chunk)]

        def ibuf_slice(p):
            return ibufs.at[pl.ds(p * rows_chunk, rows_chunk)]

        def obuf_slice(p):
            return obufs.at[pl.ds(p * rows_chunk, rows_chunk)]

        pltpu.async_copy(in_slice(0), ibuf_slice(0), isem)

        def step(k):
            p = lax.rem(k, 2)

            @pl.when(k + 1 < n_chunks)
            def _():
                pltpu.async_copy(in_slice(k + 1), ibuf_slice(1 - p), isem)

            pltpu.make_async_copy(in_slice(k), ibuf_slice(p), isem).wait()

            @pl.when(k >= 2)
            def _():
                pltpu.make_async_copy(
                    obuf_slice(p), out_slice(k - 2), osem
                ).wait()

            rbase = p * rows_chunk

            def row(r):
                rv = jnp.broadcast_to(rbase + r, (_LANES,)).astype(jnp.int32)
                for j in range(K // _LANES):
                    obufs[rbase + r, pl.ds(j * _LANES, _LANES)] = (
                        plsc.load_gather(ibufs, [rv, kvecs[j]])
                    )

            pl.loop(0, rows_chunk, unroll=4)(row)
            pltpu.async_copy(obuf_slice(p), out_slice(k), osem)

        pl.loop(0, n_chunks)(step)

        for k in (n_chunks - 2, n_chunks - 1):
            pltpu.make_async_copy(
                obufs.at[pl.ds((k % 2) * rows_chunk, rows_chunk)],
                out_slice(k),
                osem,
            ).wait()

    return gather_rows


def kernel(x, keep):
    B, C, H, W = x.shape
    K = keep.shape[0]
    P = B * H * W
    xt = x.transpose(0, 2, 3, 1).reshape(P, C)
    gather_rows = _build_gather(P, C, K, rows_chunk=56)
    out_t = gather_rows(xt, keep)
    return out_t.reshape(B, H, W, K).transpose(0, 3, 1, 2)


# speedup vs baseline: 3.0797x; 1.0053x over previous
import functools

import jax
import jax.numpy as jnp
from jax import lax
from jax.experimental import pallas as pl
from jax.experimental.pallas import tpu as pltpu
from jax.experimental.pallas import tpu_sc as plsc

_LANES = 16


def _build_gather(P, C, K, rows_chunk):
    info = plsc.get_sparse_core_info()
    nc, ns = info.num_cores, info.num_subcores
    nw = nc * ns
    rows_w = P // nw
    n_chunks = rows_w // rows_chunk
    assert P % nw == 0 and rows_w % rows_chunk == 0 and K % _LANES == 0

    mesh = plsc.VectorSubcoreMesh(core_axis_name="c", subcore_axis_name="s")

    @functools.partial(
        pl.kernel,
        mesh=mesh,
        out_type=jax.ShapeDtypeStruct((P, K), jnp.float32),
        compiler_params=pltpu.CompilerParams(needs_layout_passes=False),
        scratch_types=[
            pltpu.VMEM((K,), jnp.int32),
            pltpu.VMEM((2, rows_chunk, C), jnp.float32),
            pltpu.VMEM((2, rows_chunk, K), jnp.float32),
            pltpu.SemaphoreType.DMA,
            pltpu.SemaphoreType.DMA,
        ],
    )
    def gather_rows(x_hbm, keep_hbm, out_hbm, keep_v, ibufs, obufs, isem, osem):
        wid = lax.axis_index("s") * nc + lax.axis_index("c")
        row0 = wid * rows_w
        pltpu.sync_copy(keep_hbm, keep_v)
        kvecs = [keep_v[pl.ds(j * _LANES, _LANES)] for j in range(K // _LANES)]

        def in_slice(k):
            return x_hbm.at[pl.ds(row0 + k * rows_chunk, rows_chunk)]

        def out_slice(k):
            return out_hbm.at[pl.ds(row0 + k * rows_chunk, rows_chunk)]

        pltpu.async_copy(in_slice(0), ibufs.at[0], isem)

        def step(k):
            p = lax.rem(k, 2)

            @pl.when(k + 1 < n_chunks)
            def _():
                pltpu.async_copy(in_slice(k + 1), ibufs.at[1 - p], isem)

            pltpu.make_async_copy(in_slice(k), ibufs.at[p], isem).wait()

            @pl.when(k >= 2)
            def _():
                pltpu.make_async_copy(
                    obufs.at[p], out_slice(k - 2), osem
                ).wait()

            pv = jnp.broadcast_to(p, (_LANES,)).astype(jnp.int32)

            def row(r):
                rv = jnp.broadcast_to(r, (_LANES,)).astype(jnp.int32)
                for j in range(K // _LANES):
                    obufs[p, r, pl.ds(j * _LANES, _LANES)] = plsc.load_gather(
                        ibufs, [pv, rv, kvecs[j]]
                    )

            pl.loop(0, rows_chunk, unroll=4)(row)
            pltpu.async_copy(obufs.at[p], out_slice(k), osem)

        pl.loop(0, n_chunks)(step)

        for k in (n_chunks - 2, n_chunks - 1):
            pltpu.make_async_copy(obufs.at[k % 2], out_slice(k), osem).wait()

    return gather_rows


def kernel(x, keep):
    B, C, H, W = x.shape
    K = keep.shape[0]
    P = B * H * W
    xt = x.transpose(0, 2, 3, 1).reshape(P, C)
    gather_rows = _build_gather(P, C, K, rows_chunk=56)
    out_t = gather_rows(xt, keep)
    return out_t.reshape(B, H, W, K).transpose(0, 3, 1, 2)
